# baseline (speedup 1.0000x reference)
import functools

import jax
import jax.numpy as jnp
from jax import lax
from jax.experimental import pallas as pl
from jax.experimental.pallas import tpu as pltpu
from jax.experimental.pallas import tpu_sc as plsc

B = 16384
J = 55
NE = 21
JO = J + NE
XROWS = J * 4 * 128 * 4
OROWS = JO * 4 * 128 * 4

NC = 2
NS = 16
NW = NC * NS
COPY_ROWS = XROWS // NW
UNITS = NE * 4
UPW = 3
HALF = 256
BUF = 256
NBUF = 3


def _sc_body(x_hbm, idx_hbm, e_hbm, out_hbm, idx_v, e_v,
             b0, b1, b2, s0i, s1i, s2i, s0o, s1o, s2o):
    c = lax.axis_index("c")
    s = lax.axis_index("s")
    wid = s * NC + c

    pltpu.sync_copy(idx_hbm, idx_v)
    pltpu.sync_copy(e_hbm, e_v)

    bufs = (b0, b1, b2)
    isems = (s0i, s1i, s2i)
    osems = (s0o, s1o, s2o)

    ejobs = []
    for u in range(UPW):
        unit = lax.rem(wid + u * NW, UNITS)
        k = unit // 4
        r = unit - k * 4
        iv = idx_v[pl.ds(k, 16)]
        idxk = iv[0]
        ev = e_v[pl.ds(k * 16, 16)]
        src0 = idxk * 2048 + r * 512
        dst0 = (J + k) * 2048 + r * 512
        for h in range(2):
            ejobs.append((src0 + h * HALF, dst0 + h * HALF, HALF, ev))
    cjobs = []
    crows = [BUF] * 13 + [COPY_ROWS - 13 * BUF]
    off = 0
    for rws in crows:
        row = wid * COPY_ROWS + off
        cjobs.append((row, row, rws, None))
        off += rws
    jobs = []
    while ejobs or cjobs:
        if ejobs:
            jobs.append(ejobs.pop(0))
        for _ in range(2):
            if cjobs:
                jobs.append(cjobs.pop(0))

    n = len(jobs)
    incp = [None] * NBUF
    outcp = [None] * NBUF

    for p in range(2):
        srcp, _, rowsp, _ = jobs[p]
        incp[p] = pltpu.async_copy(
            x_hbm.at[pl.ds(srcp, rowsp)], bufs[p].at[pl.ds(0, rowsp)], isems[p])

    for i in range(n):
        bi = i % NBUF
        src, dst, rows, ev = jobs[i]
        incp[bi].wait()
        if ev is not None:
            buf = bufs[bi]
            evs = [ev[m] for m in range(16)]

            @plsc.parallel_loop(0, HALF // 4)
            def bt_body(bt, buf=buf, evs=evs):
                base = bt * 4
                rows_v = [
                    [buf[base + cp, pl.ds(l * 16, 16)] for l in range(8)]
                    for cp in range(4)
                ]
                for cc in range(4):
                    for l in range(8):
                        acc = rows_v[0][l] * evs[0 * 4 + cc]
                        for cp in range(1, 4):
                            acc = acc + rows_v[cp][l] * evs[cp * 4 + cc]
                        buf[base + cc, pl.ds(l * 16, 16)] = acc

        if i + 2 < n:
            nb = (i + 2) % NBUF
            if outcp[nb] is not None:
                outcp[nb].wait()
            nsrc, _, nrows, _ = jobs[i + 2]
            incp[nb] = pltpu.async_copy(
                x_hbm.at[pl.ds(nsrc, nrows)], bufs[nb].at[pl.ds(0, nrows)],
                isems[nb])
        outcp[bi] = pltpu.async_copy(
            bufs[bi].at[pl.ds(0, rows)], out_hbm.at[pl.ds(dst, rows)], osems[bi])

    for p in range(NBUF):
        if outcp[p] is not None:
            outcp[p].wait()


@jax.jit
def _run(x, idx_pad, e_flat):
    mesh = plsc.VectorSubcoreMesh(
        core_axis_name="c", subcore_axis_name="s", num_cores=NC, num_subcores=NS)
    return pl.kernel(
        _sc_body,
        out_type=jax.ShapeDtypeStruct((OROWS, 128), jnp.float32),
        mesh=mesh,
        scratch_types=[
            pltpu.VMEM((64,), jnp.int32),
            pltpu.VMEM((352,), jnp.float32),
            pltpu.VMEM((BUF, 128), jnp.float32),
            pltpu.VMEM((BUF, 128), jnp.float32),
            pltpu.VMEM((BUF, 128), jnp.float32),
            pltpu.SemaphoreType.DMA,
            pltpu.SemaphoreType.DMA,
            pltpu.SemaphoreType.DMA,
            pltpu.SemaphoreType.DMA,
            pltpu.SemaphoreType.DMA,
            pltpu.SemaphoreType.DMA,
        ],
        compiler_params=pltpu.CompilerParams(
            use_tc_tiling_on_sc=False, needs_layout_passes=False),
    )(x, idx_pad, e_flat)


def kernel(joints_transforms, extra_joint_parent_indices, extra_joint_transforms):
    x = (joints_transforms
         .reshape(128, 128, J, 4, 4)
         .transpose(2, 3, 0, 4, 1)
         .reshape(XROWS, 128))
    idx = extra_joint_parent_indices.astype(jnp.int32)
    idx_pad = jnp.concatenate([idx, jnp.zeros((64 - NE,), jnp.int32)])
    e_flat = jnp.concatenate(
        [extra_joint_transforms.reshape(NE * 16), jnp.zeros((16,), jnp.float32)])
    out = _run(x, idx_pad, e_flat)
    return (out
            .reshape(JO, 4, 128, 4, 128)
            .transpose(2, 4, 0, 1, 3)
            .reshape(B, JO, 4, 4))


# speedup vs baseline: 1.0322x; 1.0322x over previous
import functools

import jax
import jax.numpy as jnp
from jax import lax
from jax.experimental import pallas as pl
from jax.experimental.pallas import tpu as pltpu
from jax.experimental.pallas import tpu_sc as plsc

B = 16384
J = 55
NE = 21
JO = J + NE
XROWS = J * 4 * 128 * 4
OROWS = JO * 4 * 128 * 4

NC = 2
NS = 16
NW = NC * NS
COPY_ROWS = XROWS // NW
UNITS = NE * 4
UPW = 3
HALF = 256
BUF = 256
NBUF = 3


def _sc_body(x_hbm, idx_hbm, e_hbm, out_hbm, idx_v, e_v,
             b0, b1, b2, s0i, s1i, s2i, s0o, s1o, s2o):
    c = lax.axis_index("c")
    s = lax.axis_index("s")
    wid = s * NC + c

    pltpu.sync_copy(idx_hbm, idx_v)
    pltpu.sync_copy(e_hbm, e_v)

    bufs = (b0, b1, b2)
    isems = (s0i, s1i, s2i)
    osems = (s0o, s1o, s2o)

    ejobs = []
    for u in range(UPW):
        unit = lax.rem(wid + u * NW, UNITS)
        k = unit // 4
        r = unit - k * 4
        iv = idx_v[pl.ds(k, 16)]
        idxk = iv[0]
        ev = e_v[pl.ds(k * 16, 16)]
        src0 = idxk * 2048 + r * 512
        dst0 = (J + k) * 2048 + r * 512
        for h in range(2):
            ejobs.append((src0 + h * HALF, dst0 + h * HALF, HALF, ev))
    cjobs = []
    crows = [BUF] * 13 + [COPY_ROWS - 13 * BUF]
    off = 0
    for rws in crows:
        row = wid * COPY_ROWS + off
        cjobs.append((row, row, rws, None))
        off += rws
    jobs = []
    while ejobs or cjobs:
        if ejobs:
            jobs.append(ejobs.pop(0))
        for _ in range(2):
            if cjobs:
                jobs.append(cjobs.pop(0))

    n = len(jobs)
    incp = [None] * NBUF
    outcp = [None] * NBUF

    for p in range(2):
        srcp, _, rowsp, _ = jobs[p]
        incp[p] = pltpu.async_copy(
            x_hbm.at[pl.ds(srcp, rowsp)], bufs[p].at[pl.ds(0, rowsp)], isems[p])

    for i in range(n):
        bi = i % NBUF
        src, dst, rows, ev = jobs[i]
        incp[bi].wait()
        if ev is not None:
            buf = bufs[bi]
            evs = [ev[m] for m in range(16)]

            def bt_body(bt, carry, buf=buf, evs=evs):
                base = bt * 4
                rows_v = [
                    [buf[base + cp, pl.ds(l * 16, 16)] for l in range(8)]
                    for cp in range(4)
                ]
                for cc in range(4):
                    for l in range(8):
                        acc = rows_v[0][l] * evs[0 * 4 + cc]
                        for cp in range(1, 4):
                            acc = acc + rows_v[cp][l] * evs[cp * 4 + cc]
                        buf[base + cc, pl.ds(l * 16, 16)] = acc
                return carry

            lax.fori_loop(0, HALF // 4, bt_body, 0)

        if i + 2 < n:
            nb = (i + 2) % NBUF
            if outcp[nb] is not None:
                outcp[nb].wait()
            nsrc, _, nrows, _ = jobs[i + 2]
            incp[nb] = pltpu.async_copy(
                x_hbm.at[pl.ds(nsrc, nrows)], bufs[nb].at[pl.ds(0, nrows)],
                isems[nb])
        outcp[bi] = pltpu.async_copy(
            bufs[bi].at[pl.ds(0, rows)], out_hbm.at[pl.ds(dst, rows)], osems[bi])

    for p in range(NBUF):
        if outcp[p] is not None:
            outcp[p].wait()


@jax.jit
def _run(x, idx_pad, e_flat):
    mesh = plsc.VectorSubcoreMesh(
        core_axis_name="c", subcore_axis_name="s", num_cores=NC, num_subcores=NS)
    return pl.kernel(
        _sc_body,
        out_type=jax.ShapeDtypeStruct((OROWS, 128), jnp.float32),
        mesh=mesh,
        scratch_types=[
            pltpu.VMEM((64,), jnp.int32),
            pltpu.VMEM((352,), jnp.float32),
            pltpu.VMEM((BUF, 128), jnp.float32),
            pltpu.VMEM((BUF, 128), jnp.float32),
            pltpu.VMEM((BUF, 128), jnp.float32),
            pltpu.SemaphoreType.DMA,
            pltpu.SemaphoreType.DMA,
            pltpu.SemaphoreType.DMA,
            pltpu.SemaphoreType.DMA,
            pltpu.SemaphoreType.DMA,
            pltpu.SemaphoreType.DMA,
        ],
        compiler_params=pltpu.CompilerParams(
            use_tc_tiling_on_sc=False, needs_layout_passes=False),
    )(x, idx_pad, e_flat)


def kernel(joints_transforms, extra_joint_parent_indices, extra_joint_transforms):
    x = (joints_transforms
         .reshape(128, 128, J, 4, 4)
         .transpose(2, 3, 0, 4, 1)
         .reshape(XROWS, 128))
    idx = extra_joint_parent_indices.astype(jnp.int32)
    idx_pad = jnp.concatenate([idx, jnp.zeros((64 - NE,), jnp.int32)])
    e_flat = jnp.concatenate(
        [extra_joint_transforms.reshape(NE * 16), jnp.zeros((16,), jnp.float32)])
    out = _run(x, idx_pad, e_flat)
    return (out
            .reshape(JO, 4, 128, 4, 128)
            .transpose(2, 4, 0, 1, 3)
            .reshape(B, JO, 4, 4))
